# baseline (speedup 1.0000x reference)
import functools

import jax
import jax.numpy as jnp
from jax import lax
from jax.experimental import pallas as pl
from jax.experimental.pallas import tpu as pltpu
from jax.experimental.pallas import tpu_sc as plsc

_SC_CORES = 2
_SC_SUBCORES = 16
_NW = _SC_CORES * _SC_SUBCORES
_LANES = 16

_POSE_BLK = 128


def _tc_body(p_pad, pose_ref, rg_ref, qx_ref, qy_ref, dist_ref, idx_ref):
    x = pose_ref[:, 0:1]
    y = pose_ref[:, 1:2]
    l = pose_ref[:, 2:3]
    w = pose_ref[:, 3:4]
    yaw = pose_ref[:, 4:5]
    c = jnp.cos(yaw)
    s = jnp.sin(yaw)
    lc = l / 2 * c
    ls = l / 2 * s
    wc = w / 2 * c
    ws = w / 2 * s
    dxs = (lc + ws, lc - ws, -lc - ws, -lc + ws)
    dys = (ls - wc, ls + wc, -ls + wc, -ls - wc)
    rgx = rg_ref[0:1, :]
    rgy = rg_ref[1:2, :]
    blk = x.shape[0]
    iota_f = lax.broadcasted_iota(jnp.int32, (blk, p_pad), 1).astype(jnp.float32)
    big = jnp.float32(1e9)
    qxs, qys, dists, idxs = [], [], [], []
    for k in range(4):
        qx = dxs[k] + x
        qy = dys[k] + y
        d2 = (qx - rgx) ** 2 + (qy - rgy) ** 2
        md = jnp.min(d2, axis=1, keepdims=True)
        sel = jnp.where(d2 == md, iota_f, big)
        ix = jnp.min(sel, axis=1, keepdims=True)
        qxs.append(qx)
        qys.append(qy)
        dists.append(jnp.sqrt(md))
        idxs.append(ix)
    qx_ref[...] = jnp.concatenate(qxs, axis=1)
    qy_ref[...] = jnp.concatenate(qys, axis=1)
    dist_ref[...] = jnp.concatenate(dists, axis=1)
    idx_ref[...] = jnp.concatenate(idxs, axis=1).astype(jnp.int32)


def _run_tc(pose, rg, npose, p_pad):
    grid = npose // _POSE_BLK
    pose_spec = pl.BlockSpec((_POSE_BLK, 8), lambda i: (i, 0))
    rg_spec = pl.BlockSpec((2, p_pad), lambda i: (0, 0))
    out_spec = pl.BlockSpec((_POSE_BLK, 4), lambda i: (i, 0))
    f32 = jnp.float32
    return pl.pallas_call(
        functools.partial(_tc_body, p_pad),
        grid=(grid,),
        in_specs=[pose_spec, rg_spec],
        out_specs=[out_spec] * 4,
        out_shape=[
            jax.ShapeDtypeStruct((npose, 4), f32),
            jax.ShapeDtypeStruct((npose, 4), f32),
            jax.ShapeDtypeStruct((npose, 4), f32),
            jax.ShapeDtypeStruct((npose, 4), jnp.int32),
        ],
    )(pose, rg)


@functools.cache
def _make_sc_kernel(npose, n_points):
    poses_per_tile = npose // _NW
    qs_per_tile = poses_per_tile * 4
    n_chunks = qs_per_tile // _LANES
    mesh = plsc.VectorSubcoreMesh(core_axis_name="c", subcore_axis_name="s")

    @functools.partial(
        pl.kernel,
        mesh=mesh,
        out_type=jax.ShapeDtypeStruct((npose,), jnp.float32),
        compiler_params=pltpu.CompilerParams(needs_layout_passes=False),
        scratch_types=[
            pltpu.VMEM((qs_per_tile,), jnp.int32),
            pltpu.VMEM((qs_per_tile,), jnp.float32),
            pltpu.VMEM((qs_per_tile,), jnp.float32),
            pltpu.VMEM((qs_per_tile,), jnp.float32),
            pltpu.VMEM((4 * n_points,), jnp.float32),
            pltpu.VMEM((n_points,), jnp.int32),
            pltpu.VMEM((qs_per_tile,), jnp.float32),
            pltpu.VMEM((poses_per_tile,), jnp.float32),
        ],
    )
    def sc_kernel(idx_hbm, qx_hbm, qy_hbm, dist_hbm, tab_hbm, ids_hbm,
                  out_hbm, idx_v, qx_v, qy_v, dist_v, tab_v, ids_v,
                  signed_v, out_v):
        wid = lax.axis_index("s") * _SC_CORES + lax.axis_index("c")
        qbase = wid * qs_per_tile
        pltpu.sync_copy(idx_hbm.at[pl.ds(qbase, qs_per_tile)], idx_v)
        pltpu.sync_copy(qx_hbm.at[pl.ds(qbase, qs_per_tile)], qx_v)
        pltpu.sync_copy(qy_hbm.at[pl.ds(qbase, qs_per_tile)], qy_v)
        pltpu.sync_copy(dist_hbm.at[pl.ds(qbase, qs_per_tile)], dist_v)
        pltpu.sync_copy(tab_hbm, tab_v)
        pltpu.sync_copy(ids_hbm, ids_v)
        for ch in range(n_chunks):
            sl = pl.ds(ch * _LANES, _LANES)
            ix = idx_v[sl]
            pix = jnp.maximum(ix - 1, 0)
            nx = plsc.load_gather(tab_v, [ix])
            ny = plsc.load_gather(tab_v, [ix + n_points])
            dvx = plsc.load_gather(tab_v, [ix + 2 * n_points])
            dvy = plsc.load_gather(tab_v, [ix + 3 * n_points])
            pvx = plsc.load_gather(tab_v, [pix + 2 * n_points])
            pvy = plsc.load_gather(tab_v, [pix + 3 * n_points])
            idn = plsc.load_gather(ids_v, [ix])
            idp = plsc.load_gather(ids_v, [pix])
            ptx = qx_v[sl] - nx
            pty = qy_v[sl] - ny
            cr = ptx * dvy - pty * dvx
            crp = ptx * pvy - pty * pvx
            chosen = jnp.where((idn == idp) & (crp < cr), crp, cr)
            sgn = jnp.sign(chosen)
            sgn = jnp.where(sgn == 0.0, 1.0, sgn)
            signed_v[sl] = sgn * dist_v[sl]
        lane = lax.iota(jnp.int32, 16)
        for d in range(poses_per_tile // _LANES):
            base_i = d * 4 * _LANES + 4 * lane
            m = plsc.load_gather(signed_v, [base_i])
            for k in (1, 2, 3):
                m = jnp.maximum(m, plsc.load_gather(signed_v, [base_i + k]))
            out_v[pl.ds(d * _LANES, _LANES)] = m
        pltpu.sync_copy(
            out_v, out_hbm.at[pl.ds(wid * poses_per_tile, poses_per_tile)])

    return sc_kernel


def kernel(traj_pred, agents, agents_mask, rg_xy, rg_dir_xy, rg_ids):
    weight = 0.1
    B, A, T, _ = traj_pred.shape
    P = rg_xy.shape[0]
    n = B * A * T
    npose_pad = -(-n // 1024) * 1024
    half = npose_pad // 2
    p_pad = -(-P // 128) * 128

    x = traj_pred[..., 0].reshape(-1)
    y = traj_pred[..., 1].reshape(-1)
    yaw = traj_pred[..., 2].reshape(-1)
    l = jnp.broadcast_to(agents[:, :, -1, 5][..., None], (B, A, T)).reshape(-1)
    w = jnp.broadcast_to(agents[:, :, -1, 6][..., None], (B, A, T)).reshape(-1)
    zero = jnp.zeros_like(x)
    pose = jnp.stack([x, y, l, w, yaw, zero, zero, zero], axis=-1)
    pose = jnp.pad(pose, ((0, npose_pad - n), (0, 0)))

    rg = jnp.pad(rg_xy.T, ((0, 0), (0, p_pad - P)), constant_values=1e30)

    tab = jnp.concatenate(
        [rg_xy[:, 0], rg_xy[:, 1], rg_dir_xy[:, 0], rg_dir_xy[:, 1]])
    ids = rg_ids.astype(jnp.int32)

    sc = _make_sc_kernel(half, P)
    halves = []
    for h in range(2):
        qx4, qy4, dist4, idx4 = _run_tc(
            lax.slice_in_dim(pose, h * half, (h + 1) * half, axis=0),
            rg, half, p_pad)
        halves.append(sc(idx4.reshape(-1), qx4.reshape(-1), qy4.reshape(-1),
                         dist4.reshape(-1), tab, ids))
    pose_signed = jnp.concatenate(halves)

    pm = pose_signed[:n].reshape(B, A, T)
    pm = pm * (pm[:, :, 0:1] < 0)
    cost = jax.nn.relu(pm)
    cost = cost * (~agents_mask)[:, :, None] * weight
    return -cost


# speedup vs baseline: 1.1187x; 1.1187x over previous
import functools

import jax
import jax.numpy as jnp
from jax import lax
from jax.experimental import pallas as pl
from jax.experimental.pallas import tpu as pltpu
from jax.experimental.pallas import tpu_sc as plsc

_SC_CORES = 2
_SC_SUBCORES = 16
_NW = _SC_CORES * _SC_SUBCORES
_LANES = 16

_POSE_BLK = 256


def _tc_body(p_pad, pose_ref, rg_ref, out_ref):
    x = pose_ref[:, 0:1]
    y = pose_ref[:, 1:2]
    l = pose_ref[:, 2:3]
    w = pose_ref[:, 3:4]
    yaw = pose_ref[:, 4:5]
    c = jnp.cos(yaw)
    s = jnp.sin(yaw)
    lc = l / 2 * c
    ls = l / 2 * s
    wc = w / 2 * c
    ws = w / 2 * s
    dxs = (lc + ws, lc - ws, -lc - ws, -lc + ws)
    dys = (ls - wc, ls + wc, -ls + wc, -ls - wc)
    rgx = rg_ref[0:1, :]
    rgy = rg_ref[1:2, :]
    blk = x.shape[0]
    iota_f = lax.broadcasted_iota(jnp.int32, (blk, p_pad), 1).astype(jnp.float32)
    big = jnp.float32(1e9)
    qxs, qys, dists, idxs = [], [], [], []
    for k in range(4):
        qx = dxs[k] + x
        qy = dys[k] + y
        d2 = (qx - rgx) ** 2 + (qy - rgy) ** 2
        md = jnp.min(d2, axis=1, keepdims=True)
        sel = jnp.where(d2 == md, iota_f, big)
        ix = jnp.min(sel, axis=1, keepdims=True)
        qxs.append(qx)
        qys.append(qy)
        dists.append(jnp.sqrt(md))
        idxs.append(ix)
    out_ref[...] = jnp.concatenate(qxs + qys + dists + idxs, axis=1)


def _run_tc(pose, rg, npose, p_pad):
    grid = npose // _POSE_BLK
    pose_spec = pl.BlockSpec((_POSE_BLK, 8), lambda i: (i, 0))
    rg_spec = pl.BlockSpec((2, p_pad), lambda i: (0, 0))
    out_spec = pl.BlockSpec((_POSE_BLK, 16), lambda i: (i, 0))
    return pl.pallas_call(
        functools.partial(_tc_body, p_pad),
        grid=(grid,),
        in_specs=[pose_spec, rg_spec],
        out_specs=out_spec,
        out_shape=jax.ShapeDtypeStruct((npose, 16), jnp.float32),
    )(pose, rg)


@functools.cache
def _make_sc_kernel(npose, n_points):
    poses_per_tile = npose // _NW
    qs_per_tile = poses_per_tile * 4
    n_chunks = qs_per_tile // _LANES
    mesh = plsc.VectorSubcoreMesh(core_axis_name="c", subcore_axis_name="s")

    @functools.partial(
        pl.kernel,
        mesh=mesh,
        out_type=jax.ShapeDtypeStruct((npose,), jnp.float32),
        compiler_params=pltpu.CompilerParams(needs_layout_passes=False),
        scratch_types=[
            pltpu.VMEM((16 * poses_per_tile,), jnp.float32),
            pltpu.VMEM((4 * n_points,), jnp.float32),
            pltpu.VMEM((n_points,), jnp.int32),
            pltpu.VMEM((qs_per_tile,), jnp.float32),
            pltpu.VMEM((poses_per_tile,), jnp.float32),
            pltpu.SemaphoreType.DMA,
        ],
    )
    def sc_kernel(q_hbm, tab_hbm, ids_hbm,
                  out_hbm, q_v, tab_v, ids_v, signed_v, out_v, sem):
        wid = lax.axis_index("s") * _SC_CORES + lax.axis_index("c")
        copies = [
            pltpu.async_copy(
                q_hbm.at[pl.ds(wid * 16 * poses_per_tile,
                               16 * poses_per_tile)], q_v, sem),
            pltpu.async_copy(tab_hbm, tab_v, sem),
            pltpu.async_copy(ids_hbm, ids_v, sem),
        ]
        for cp in copies:
            cp.wait()
        lane0 = lax.iota(jnp.int32, 16)
        fbase = 16 * (lane0 >> 2) + (lane0 & 3)
        for ch in range(n_chunks):
            sl = pl.ds(ch * _LANES, _LANES)
            fb = fbase + 64 * ch
            qx = plsc.load_gather(q_v, [fb])
            qy = plsc.load_gather(q_v, [fb + 4])
            dist = plsc.load_gather(q_v, [fb + 8])
            ix = plsc.load_gather(q_v, [fb + 12]).astype(jnp.int32)
            pix = jnp.maximum(ix - 1, 0)
            nx = plsc.load_gather(tab_v, [ix])
            ny = plsc.load_gather(tab_v, [ix + n_points])
            dvx = plsc.load_gather(tab_v, [ix + 2 * n_points])
            dvy = plsc.load_gather(tab_v, [ix + 3 * n_points])
            pvx = plsc.load_gather(tab_v, [pix + 2 * n_points])
            pvy = plsc.load_gather(tab_v, [pix + 3 * n_points])
            idn = plsc.load_gather(ids_v, [ix])
            idp = plsc.load_gather(ids_v, [pix])
            ptx = qx - nx
            pty = qy - ny
            cr = ptx * dvy - pty * dvx
            crp = ptx * pvy - pty * pvx
            chosen = jnp.where((idn == idp) & (crp < cr), crp, cr)
            sgn = jnp.sign(chosen)
            sgn = jnp.where(sgn == 0.0, 1.0, sgn)
            signed_v[sl] = sgn * dist
        lane = lax.iota(jnp.int32, 16)
        for d in range(poses_per_tile // _LANES):
            base_i = d * 4 * _LANES + 4 * lane
            m = plsc.load_gather(signed_v, [base_i])
            for k in (1, 2, 3):
                m = jnp.maximum(m, plsc.load_gather(signed_v, [base_i + k]))
            out_v[pl.ds(d * _LANES, _LANES)] = m
        pltpu.sync_copy(
            out_v, out_hbm.at[pl.ds(wid * poses_per_tile, poses_per_tile)])

    return sc_kernel


def kernel(traj_pred, agents, agents_mask, rg_xy, rg_dir_xy, rg_ids):
    weight = 0.1
    B, A, T, _ = traj_pred.shape
    P = rg_xy.shape[0]
    n = B * A * T
    npose_pad = -(-n // 1024) * 1024
    p_pad = -(-P // 128) * 128

    x = traj_pred[..., 0].reshape(-1)
    y = traj_pred[..., 1].reshape(-1)
    yaw = traj_pred[..., 2].reshape(-1)
    l = jnp.broadcast_to(agents[:, :, -1, 5][..., None], (B, A, T)).reshape(-1)
    w = jnp.broadcast_to(agents[:, :, -1, 6][..., None], (B, A, T)).reshape(-1)
    zero = jnp.zeros_like(x)
    pose = jnp.stack([x, y, l, w, yaw, zero, zero, zero], axis=-1)
    pose = jnp.pad(pose, ((0, npose_pad - n), (0, 0)))

    rg = jnp.pad(rg_xy.T, ((0, 0), (0, p_pad - P)), constant_values=1e30)

    tab = jnp.concatenate(
        [rg_xy[:, 0], rg_xy[:, 1], rg_dir_xy[:, 0], rg_dir_xy[:, 1]])
    ids = rg_ids.astype(jnp.int32)

    packed = _run_tc(pose, rg, npose_pad, p_pad)
    sc = _make_sc_kernel(npose_pad, P)
    pose_signed = sc(packed.reshape(-1), tab, ids)

    pm = pose_signed[:n].reshape(B, A, T)
    pm = pm * (pm[:, :, 0:1] < 0)
    cost = jax.nn.relu(pm)
    cost = cost * (~agents_mask)[:, :, None] * weight
    return -cost
